# Initial kernel scaffold; baseline (speedup 1.0000x reference)
import jax
import jax.numpy as jnp
from jax.experimental import pallas as pl


def kernel(z, weight):
    raise NotImplementedError("write your pallas kernel here")


# speedup vs baseline: 6.8895x; 6.8895x over previous
import functools

import jax
import jax.numpy as jnp
from jax.experimental import pallas as pl
from jax.experimental.pallas import tpu as pltpu

_N_E = 8192
_E_DIM = 32
_BETA = 0.25
_N_TOK = 8192
_BLK = 256
_NBLK = _N_TOK // _BLK


def _vq_body(z_ref, w_ref, zsq_ref, wsq_ref,
             oh_ref, zq_ref, idx_ref, loss_ref, perp_ref,
             counts_ref, lsum_ref):
    i = pl.program_id(0)
    zb = z_ref[...]
    w = w_ref[...]
    zsq = zsq_ref[...]
    wsq = wsq_ref[...]

    mm = jax.lax.dot_general(
        zb, w, dimension_numbers=(((1,), (1,)), ((), ())),
        preferred_element_type=jnp.float32)
    d = zsq + wsq - 2.0 * mm

    half = _N_E // 2
    iota = jax.lax.broadcasted_iota(jnp.int32, (_BLK, _N_E), 1)
    d_a, d_b = d[:, :half], d[:, half:]
    i_a, i_b = iota[:, :half], iota[:, half:]
    v_a = jnp.min(d_a, axis=1, keepdims=True)
    j_a = jnp.min(jnp.where(d_a == v_a, i_a, _N_E), axis=1, keepdims=True)
    v_b = jnp.min(d_b, axis=1, keepdims=True)
    j_b = jnp.min(jnp.where(d_b == v_b, i_b, _N_E), axis=1, keepdims=True)
    v_a16 = v_a.astype(jnp.bfloat16).astype(jnp.float32)
    keep_a = (v_a16 < v_b) | ((v_a16 == v_b) & (j_a < j_b))
    idx = jnp.where(keep_a, j_a, j_b)

    oh = jnp.where(iota == idx, 1.0, 0.0).astype(jnp.float32)
    oh_ref[...] = oh
    idx_ref[...] = idx

    zq = jax.lax.dot_general(
        oh, w, dimension_numbers=(((1,), (0,)), ((), ())),
        preferred_element_type=jnp.float32)
    zq_ref[...] = zq

    diff = zq - zb
    part_loss = jnp.sum(diff * diff)
    part_counts = jnp.sum(oh, axis=0, keepdims=True)

    @pl.when(i == 0)
    def _init():
        lsum_ref[0, 0] = part_loss
        counts_ref[...] = part_counts

    @pl.when(i > 0)
    def _acc():
        lsum_ref[0, 0] += part_loss
        counts_ref[...] += part_counts

    @pl.when(i == _NBLK - 1)
    def _fini():
        mse = lsum_ref[0, 0] / (_N_TOK * _E_DIM)
        loss_ref[...] = jnp.reshape(_BETA * mse + mse, (1, 1))
        e_mean = counts_ref[...] / _N_TOK
        ent = jnp.sum(e_mean * jnp.log(e_mean + 1e-10), axis=1, keepdims=True)
        perp_ref[...] = jnp.exp(-ent)


@functools.partial(jax.jit, static_argnames=())
def kernel(z, weight):
    zsq = jnp.sum(z ** 2, axis=1, keepdims=True)
    wsq = jnp.sum(weight ** 2, axis=1)[None, :]

    oh, zq, idx, loss, perp = pl.pallas_call(
        _vq_body,
        grid=(_NBLK,),
        in_specs=[
            pl.BlockSpec((_BLK, _E_DIM), lambda i: (i, 0)),
            pl.BlockSpec((_N_E, _E_DIM), lambda i: (0, 0)),
            pl.BlockSpec((_BLK, 1), lambda i: (i, 0)),
            pl.BlockSpec((1, _N_E), lambda i: (0, 0)),
        ],
        out_specs=[
            pl.BlockSpec((_BLK, _N_E), lambda i: (i, 0)),
            pl.BlockSpec((_BLK, _E_DIM), lambda i: (i, 0)),
            pl.BlockSpec((_BLK, 1), lambda i: (i, 0)),
            pl.BlockSpec((1, 1), lambda i: (0, 0)),
            pl.BlockSpec((1, 1), lambda i: (0, 0)),
        ],
        out_shape=[
            jax.ShapeDtypeStruct((_N_TOK, _N_E), jnp.float32),
            jax.ShapeDtypeStruct((_N_TOK, _E_DIM), jnp.float32),
            jax.ShapeDtypeStruct((_N_TOK, 1), jnp.int32),
            jax.ShapeDtypeStruct((1, 1), jnp.float32),
            jax.ShapeDtypeStruct((1, 1), jnp.float32),
        ],
        scratch_shapes=[
            pltpu.VMEM((1, _N_E), jnp.float32),
            pltpu.SMEM((1, 1), jnp.float32),
        ],
    )(z, weight, zsq, wsq)

    z_q = zq
    loss = loss[0, 0]
    perplexity = perp[0, 0]
    return (z_q, loss, (perplexity, oh, idx))
